# baseline (speedup 1.0000x reference)
import jax
import jax.numpy as jnp
from jax.experimental import pallas as pl

_BM = 8


def _dice_body(inp_ref, tgt_ref, acc_ref):
    i = pl.program_id(0)
    x = jax.nn.sigmoid(inp_ref[...])
    t = tgt_ref[...]
    num = jnp.sum(x * t, axis=1)
    den = jnp.sum(x, axis=1) + jnp.sum(t, axis=1)
    loss = 1.0 - (2.0 * num + 1.0) / (den + 1.0)
    s = jnp.sum(loss).reshape(1, 1)

    @pl.when(i == 0)
    def _init():
        acc_ref[...] = s

    @pl.when(i > 0)
    def _accum():
        acc_ref[...] += s


def kernel(inputs, targets, num_boxes):
    n_masks, n_points = inputs.shape
    total = pl.pallas_call(
        _dice_body,
        grid=(n_masks // _BM,),
        in_specs=[
            pl.BlockSpec((_BM, n_points), lambda i: (i, 0)),
            pl.BlockSpec((_BM, n_points), lambda i: (i, 0)),
        ],
        out_specs=pl.BlockSpec((1, 1), lambda i: (0, 0)),
        out_shape=jax.ShapeDtypeStruct((1, 1), jnp.float32),
    )(inputs, targets)
    return total[0, 0] / (num_boxes + 1e-6)
